# baseline (speedup 1.0000x reference)
import functools

import jax
import jax.numpy as jnp
from jax import lax
from jax.experimental import pallas as pl
from jax.experimental.pallas import tpu as pltpu
from jax.experimental.pallas import tpu_sc as plsc

N = 50000
E = 800000
IN_DIM = 6
HID = 64
HH = HID // 2

NC = 2
NS = 16

CH = 128
GCH = 8
CPS = 400
NG = CPS // GCH
EPAD = NS * CPS * CH
TRASH = N
NPAD = 50048
RPT = NPAD // NS
NPAD_D = 51200
RPT_D = NPAD_D // NS

BR = 5000
GRID = N // BR

DEPTH = 6

_mesh = plsc.VectorSubcoreMesh(
    core_axis_name="c", subcore_axis_name="s", num_cores=NC, num_subcores=NS
)


DGRP = (CPS // NC) // GCH


@functools.partial(
    pl.kernel,
    out_type=jax.ShapeDtypeStruct((NC, NPAD_D), jnp.float32),
    mesh=_mesh,
    scratch_types=[
        pltpu.VMEM((GCH, CH), jnp.int32),
        pltpu.VMEM((GCH, CH), jnp.int32),
        pltpu.VMEM((NPAD_D,), jnp.float32),
        pltpu.VMEM((RPT_D,), jnp.float32),
        pltpu.VMEM((RPT_D,), jnp.float32),
        pltpu.VMEM_SHARED((NS, NPAD_D), jnp.float32),
        pltpu.SemaphoreType.DMA,
        pltpu.SemaphoreType.DMA,
    ],
    compiler_params=pltpu.CompilerParams(
        use_tc_tiling_on_sc=False, needs_layout_passes=False
    ),
)
def _deg_kernel(dst3, deg_out, didx0, didx1, hist, racc, rtmp, slab,
                semI0, semI1):
    c = lax.axis_index("c")
    s = lax.axis_index("s")
    zero16 = jnp.zeros((16,), jnp.float32)
    one16 = jnp.ones((16,), jnp.float32)
    didx = (didx0, didx1)
    semI = (semI0, semI1)

    def zf(i, _):
        hist[pl.ds(i * 16, 16)] = zero16
        return 0

    lax.fori_loop(0, NPAD_D // 16, zf, 0)

    gbase = c * DGRP

    def load_grp(g, p):
        pltpu.async_copy(dst3.at[s, pl.ds((gbase + g) * GCH, GCH)], didx[p],
                         semI[p])

    def wait_grp(p):
        pltpu.make_async_copy(dst3.at[s, pl.ds(0, GCH)], didx[p],
                              semI[p]).wait()

    def process(p):
        for r in range(GCH):
            for i in range(CH // 16):
                v = didx[p][r, pl.ds(i * 16, 16)]
                plsc.addupdate_scatter(hist, [v], one16)

    load_grp(jnp.int32(0), 0)
    load_grp(jnp.int32(1), 1)

    def hbody(u, _):
        wait_grp(0)
        process(0)
        load_grp(2 * u + 2, 0)
        wait_grp(1)
        process(1)

        @pl.when(u < 11)
        def _():
            load_grp(2 * u + 3, 1)

        return 0

    lax.fori_loop(0, (DGRP - 1) // 2, hbody, 0)
    wait_grp(0)
    process(0)

    pltpu.sync_copy(hist, slab.at[s])
    plsc.subcore_barrier()
    rbase = s * RPT_D
    pltpu.sync_copy(slab.at[0, pl.ds(rbase, RPT_D)], racc)

    def radd(k, _):
        pltpu.sync_copy(slab.at[k, pl.ds(rbase, RPT_D)], rtmp)

        def vadd(i, _):
            racc[pl.ds(i * 16, 16)] = (
                racc[pl.ds(i * 16, 16)] + rtmp[pl.ds(i * 16, 16)]
            )
            return 0

        lax.fori_loop(0, RPT_D // 16, vadd, 0)
        return 0

    lax.fori_loop(1, NS, radd, 0)
    pltpu.sync_copy(racc, deg_out.at[c, pl.ds(rbase, RPT_D)])


@functools.partial(
    pl.kernel,
    out_type=jax.ShapeDtypeStruct((NC, NPAD, HH), jnp.float32),
    mesh=_mesh,
    scratch_types=(
        [pltpu.VMEM((CH, HH), jnp.float32)] * DEPTH
        + [pltpu.VMEM((GCH, CH), jnp.int32)] * 3
        + [pltpu.VMEM((GCH, CH), jnp.int32)] * 3
        + [pltpu.VMEM_SHARED((NPAD, HH), jnp.float32)]
        + [pltpu.SemaphoreType.DMA] * (2 * DEPTH + 3)
    ),
    compiler_params=pltpu.CompilerParams(use_tc_tiling_on_sc=False),
)
def _prop_kernel(table, src3, dst3, out, *refs):
    rows = refs[0:DEPTH]
    sidx = refs[DEPTH:DEPTH + 3]
    didx = refs[DEPTH + 3:DEPTH + 6]
    acc = refs[DEPTH + 6]
    semG = refs[DEPTH + 7:DEPTH + 7 + DEPTH]
    semS = refs[DEPTH + 7 + DEPTH:DEPTH + 7 + 2 * DEPTH]
    semI = refs[DEPTH + 7 + 2 * DEPTH:]
    c = lax.axis_index("c")
    s = lax.axis_index("s")

    zero16 = jnp.zeros((16,), jnp.float32)

    def zf(r, _):
        rows[0][r, pl.ds(0, 16)] = zero16
        rows[0][r, pl.ds(16, 16)] = zero16
        return 0

    lax.fori_loop(0, CH, zf, 0)
    base = s * RPT

    def zacc(i, _):
        pltpu.sync_copy(rows[0], acc.at[pl.ds(base + i * CH, CH)])
        return 0

    lax.fori_loop(0, RPT // CH, zacc, 0)
    _REM = RPT - (RPT // CH) * CH
    if _REM:
        pltpu.sync_copy(rows[0].at[pl.ds(0, _REM)],
                        acc.at[pl.ds(base + RPT - _REM, _REM)])
    plsc.subcore_barrier()

    def load_idx(g, p):
        pltpu.async_copy(src3.at[c, s, pl.ds(g * GCH, GCH)], sidx[p], semI[p])
        pltpu.async_copy(dst3.at[s, pl.ds(g * GCH, GCH)], didx[p], semI[p])

    def wait_idx(p):
        pltpu.make_async_copy(src3.at[c, s, pl.ds(0, GCH)], sidx[p],
                              semI[p]).wait()
        pltpu.make_async_copy(dst3.at[s, pl.ds(0, GCH)], didx[p],
                              semI[p]).wait()

    def start_g(slot, p, row, _g=None, _b=None):
        pltpu.async_copy(table.at[sidx[p].at[row]], rows[slot], semG[slot])

    def wait_g(slot):
        pltpu.make_async_copy(table.at[sidx[0].at[0]], rows[slot],
                              semG[slot]).wait()

    def start_s(slot, p, row):
        pltpu.async_copy(rows[slot], acc.at[didx[p].at[row]], semS[slot],
                         add=True)

    def wait_s(slot):
        pltpu.make_async_copy(rows[slot], acc.at[didx[0].at[0]],
                              semS[slot]).wait()

    AHEAD = 5
    SLACK = DEPTH - AHEAD
    _OFF = {0: 0, 1: (8 * 1) % DEPTH, 2: (8 * 2) % DEPTH}

    def group_body(g, pg, pn, first, last):
        off = _OFF[pg]
        if not last:
            load_idx(g + 1, pn)
        for b in range(GCH):
            slot = (off + b) % DEPTH
            s2 = (off + b + AHEAD) % DEPTH
            if b == GCH - AHEAD and not last:
                wait_idx(pn)
            wait_g(slot)
            start_s(slot, pg, b)
            if not (first and b < SLACK):
                wait_s(s2)
            if b < GCH - AHEAD:
                start_g(s2, pg, b + AHEAD)
            elif not last:
                start_g(s2, pn, b - (GCH - AHEAD))

    load_idx(jnp.int32(0), 0)
    wait_idx(0)
    for b in range(AHEAD):
        start_g(b, 0, b)
    group_body(jnp.int32(0), 0, 1, first=True, last=False)
    group_body(jnp.int32(1), 1, 2, first=False, last=False)

    def triple(u, _):
        gbase = 3 * u + 2
        group_body(gbase, 2, 0, first=False, last=False)
        group_body(gbase + 1, 0, 1, first=False, last=False)
        group_body(gbase + 2, 1, 2, first=False, last=False)
        return 0

    lax.fori_loop(0, 15, triple, 0)

    group_body(jnp.int32(47), 2, 0, first=False, last=False)
    group_body(jnp.int32(48), 0, 1, first=False, last=False)
    group_body(jnp.int32(49), 1, 2, first=False, last=True)
    for b in range(GCH - SLACK, GCH):
        wait_s((_OFF[1] + b) % DEPTH)

    plsc.subcore_barrier()
    pltpu.sync_copy(acc.at[pl.ds(base, RPT)], out.at[c, pl.ds(base, RPT)])


def _tcA1_body(x_ref, w1_ref, g_ref):
    g_ref[...] = jnp.dot(x_ref[...], w1_ref[...],
                         preferred_element_type=jnp.float32)


def _tcA1(x, W1):
    return pl.pallas_call(
        _tcA1_body,
        grid=(GRID,),
        in_specs=[
            pl.BlockSpec((BR, IN_DIM), lambda i: (i, 0)),
            pl.BlockSpec((IN_DIM, HID), lambda i: (0, 0)),
        ],
        out_specs=pl.BlockSpec((BR, HID), lambda i: (i, 0)),
        out_shape=jax.ShapeDtypeStruct((N, HID), jnp.float32),
    )(x, W1)


def _tcA_body(deg_ref, g_ref, p_ref, dinv_ref):
    deg = deg_ref[:, 0] + deg_ref[:, 1] + 1.0
    dinv = 1.0 / jnp.sqrt(deg)
    p = g_ref[...] * dinv[:, None]
    p_ref[0] = p[:, :HH]
    p_ref[1] = p[:, HH:]
    dinv_ref[...] = dinv[:, None]


def _tcA(deg2, g):
    return pl.pallas_call(
        _tcA_body,
        grid=(GRID,),
        in_specs=[
            pl.BlockSpec((BR, NC), lambda i: (i, 0)),
            pl.BlockSpec((BR, HID), lambda i: (i, 0)),
        ],
        out_specs=[
            pl.BlockSpec((NC, BR, HH), lambda i: (0, i, 0)),
            pl.BlockSpec((BR, 1), lambda i: (i, 0)),
        ],
        out_shape=[
            jax.ShapeDtypeStruct((NC, NPAD, HH), jnp.float32),
            jax.ShapeDtypeStruct((NPAD, 1), jnp.float32),
        ],
    )(deg2, g)


def _tcB_body(acc_ref, p_ref, dinv_ref, w2_ref, b1_ref, p2_ref):
    dinv = dinv_ref[...]
    hl = (acc_ref[0] + p_ref[0]) * dinv + b1_ref[:, :HH]
    hr = (acc_ref[1] + p_ref[1]) * dinv + b1_ref[:, HH:]
    h = jnp.concatenate([hl, hr], axis=1)
    h = jnp.maximum(h, 0.0)
    g2 = jnp.dot(h, w2_ref[...], preferred_element_type=jnp.float32)
    p2 = g2 * dinv
    p2_ref[0] = p2[:, :HH]
    p2_ref[1] = p2[:, HH:]


def _tcB(acc1, p, dinv, W2, b1r):
    return pl.pallas_call(
        _tcB_body,
        grid=(GRID,),
        in_specs=[
            pl.BlockSpec((NC, BR, HH), lambda i: (0, i, 0)),
            pl.BlockSpec((NC, BR, HH), lambda i: (0, i, 0)),
            pl.BlockSpec((BR, 1), lambda i: (i, 0)),
            pl.BlockSpec((HID, HID), lambda i: (0, 0)),
            pl.BlockSpec((1, HID), lambda i: (0, 0)),
        ],
        out_specs=pl.BlockSpec((NC, BR, HH), lambda i: (0, i, 0)),
        out_shape=jax.ShapeDtypeStruct((NC, NPAD, HH), jnp.float32),
    )(acc1, p, dinv, W2, b1r)


def _tcC_body(acc_ref, p2_ref, dinv_ref, whw_ref, b2_ref, bhw_ref, out_ref):
    dinv = dinv_ref[...]
    hl = (acc_ref[0] + p2_ref[0]) * dinv + b2_ref[:, :HH]
    hr = (acc_ref[1] + p2_ref[1]) * dinv + b2_ref[:, HH:]
    h = jnp.concatenate([hl, hr], axis=1)
    h = jnp.maximum(h, 0.0)
    out_ref[...] = (
        jnp.dot(h, whw_ref[...], preferred_element_type=jnp.float32)
        + bhw_ref[...]
    )


def _tcC(acc2, p2, dinv, Whw, b2r, bhw):
    return pl.pallas_call(
        _tcC_body,
        grid=(GRID,),
        in_specs=[
            pl.BlockSpec((NC, BR, HH), lambda i: (0, i, 0)),
            pl.BlockSpec((NC, BR, HH), lambda i: (0, i, 0)),
            pl.BlockSpec((BR, 1), lambda i: (i, 0)),
            pl.BlockSpec((HID, 2), lambda i: (0, 0)),
            pl.BlockSpec((1, HID), lambda i: (0, 0)),
            pl.BlockSpec((1, 2), lambda i: (0, 0)),
        ],
        out_specs=pl.BlockSpec((BR, 2), lambda i: (i, 0)),
        out_shape=jax.ShapeDtypeStruct((N, 2), jnp.float32),
    )(acc2, p2, dinv, Whw, b2r, bhw)


def kernel(x, edge_index, W1, b1, W2, b2, Wo, bo, Ww, bw):
    ei = edge_index.astype(jnp.int32)
    pad = jnp.full((EPAD - E,), TRASH, jnp.int32)
    src_p = jnp.concatenate([ei[0], pad])
    src3 = jnp.stack([src_p, src_p + NPAD]).reshape(NC, NS, CPS, CH)
    dst3 = jnp.concatenate([ei[1], pad]).reshape(NS, CPS, CH)

    g1 = _tcA1(x, W1)
    deg2 = _deg_kernel(dst3)
    p1, dinv = _tcA(deg2.T, g1)
    table1 = p1.reshape(NC * NPAD, HH)
    acc1 = _prop_kernel(table1, src3, dst3)
    p2 = _tcB(acc1, p1, dinv, W2, b1.reshape(1, HID))
    table2 = p2.reshape(NC * NPAD, HH)
    acc2 = _prop_kernel(table2, src3, dst3)
    Whw = jnp.concatenate([Wo, Ww], axis=1)
    bhw = jnp.stack([bo[0], bw[0]]).reshape(1, 2)
    out = _tcC(acc2, p2, dinv, Whw, b2.reshape(1, HID), bhw)
    return out[:, 0], out[:, 1]


# speedup vs baseline: 1.0613x; 1.0613x over previous
import functools

import jax
import jax.numpy as jnp
from jax import lax
from jax.experimental import pallas as pl
from jax.experimental.pallas import tpu as pltpu
from jax.experimental.pallas import tpu_sc as plsc

N = 50000
E = 800000
IN_DIM = 6
HID = 64
HH = HID // 2

NC = 2
NS = 16

CH = 128
GCH = 8
CPS = 400
NG = CPS // GCH
EPAD = NS * CPS * CH
TRASH = N
NPAD = 50048
RPT = NPAD // NS
NPAD_D = 51200
RPT_D = NPAD_D // NS

BR = 5000
GRID = N // BR

DEPTH = 6

_mesh = plsc.VectorSubcoreMesh(
    core_axis_name="c", subcore_axis_name="s", num_cores=NC, num_subcores=NS
)


DGRP = (CPS // NC) // GCH


@functools.partial(
    pl.kernel,
    out_type=jax.ShapeDtypeStruct((NC, NPAD_D), jnp.float32),
    mesh=_mesh,
    scratch_types=[
        pltpu.VMEM((GCH, CH), jnp.int32),
        pltpu.VMEM((GCH, CH), jnp.int32),
        pltpu.VMEM((NPAD_D,), jnp.float32),
        pltpu.VMEM((RPT_D,), jnp.float32),
        pltpu.VMEM((RPT_D,), jnp.float32),
        pltpu.VMEM_SHARED((NS, NPAD_D), jnp.float32),
        pltpu.SemaphoreType.DMA,
        pltpu.SemaphoreType.DMA,
    ],
    compiler_params=pltpu.CompilerParams(
        use_tc_tiling_on_sc=False, needs_layout_passes=False
    ),
)
def _deg_kernel(dst3, deg_out, didx0, didx1, hist, racc, rtmp, slab,
                semI0, semI1):
    c = lax.axis_index("c")
    s = lax.axis_index("s")
    zero16 = jnp.zeros((16,), jnp.float32)
    one16 = jnp.ones((16,), jnp.float32)
    didx = (didx0, didx1)
    semI = (semI0, semI1)

    def zf(i, _):
        hist[pl.ds(i * 16, 16)] = zero16
        return 0

    lax.fori_loop(0, NPAD_D // 16, zf, 0)

    gbase = c * DGRP

    def load_grp(g, p):
        pltpu.async_copy(dst3.at[s, pl.ds((gbase + g) * GCH, GCH)], didx[p],
                         semI[p])

    def wait_grp(p):
        pltpu.make_async_copy(dst3.at[s, pl.ds(0, GCH)], didx[p],
                              semI[p]).wait()

    def process(p):
        for r in range(GCH):
            for i in range(CH // 16):
                v = didx[p][r, pl.ds(i * 16, 16)]
                plsc.addupdate_scatter(hist, [v], one16)

    load_grp(jnp.int32(0), 0)
    load_grp(jnp.int32(1), 1)

    def hbody(u, _):
        wait_grp(0)
        process(0)
        load_grp(2 * u + 2, 0)
        wait_grp(1)
        process(1)

        @pl.when(u < 11)
        def _():
            load_grp(2 * u + 3, 1)

        return 0

    lax.fori_loop(0, (DGRP - 1) // 2, hbody, 0)
    wait_grp(0)
    process(0)

    pltpu.sync_copy(hist, slab.at[s])
    plsc.subcore_barrier()
    rbase = s * RPT_D
    pltpu.sync_copy(slab.at[0, pl.ds(rbase, RPT_D)], racc)

    def radd(k, _):
        pltpu.sync_copy(slab.at[k, pl.ds(rbase, RPT_D)], rtmp)

        def vadd(i, _):
            racc[pl.ds(i * 16, 16)] = (
                racc[pl.ds(i * 16, 16)] + rtmp[pl.ds(i * 16, 16)]
            )
            return 0

        lax.fori_loop(0, RPT_D // 16, vadd, 0)
        return 0

    lax.fori_loop(1, NS, radd, 0)
    pltpu.sync_copy(racc, deg_out.at[c, pl.ds(rbase, RPT_D)])


@functools.partial(
    pl.kernel,
    out_type=jax.ShapeDtypeStruct((NC, NPAD, HH), jnp.float32),
    mesh=_mesh,
    scratch_types=(
        [pltpu.VMEM((CH, HH), jnp.float32)] * DEPTH
        + [pltpu.VMEM((GCH, CH), jnp.int32)] * 3
        + [pltpu.VMEM((GCH, CH), jnp.int32)] * 3
        + [pltpu.VMEM_SHARED((NPAD, HH), jnp.float32)]
        + [pltpu.SemaphoreType.DMA] * (2 * DEPTH + 3)
    ),
    compiler_params=pltpu.CompilerParams(use_tc_tiling_on_sc=False),
)
def _prop_kernel(table, src3, dst3, out, *refs):
    rows = refs[0:DEPTH]
    sidx = refs[DEPTH:DEPTH + 3]
    didx = refs[DEPTH + 3:DEPTH + 6]
    acc = refs[DEPTH + 6]
    semG = refs[DEPTH + 7:DEPTH + 7 + DEPTH]
    semS = refs[DEPTH + 7 + DEPTH:DEPTH + 7 + 2 * DEPTH]
    semI = refs[DEPTH + 7 + 2 * DEPTH:]
    c = lax.axis_index("c")
    s = lax.axis_index("s")

    def load_idx(g, p):
        pltpu.async_copy(src3.at[c, s, pl.ds(g * GCH, GCH)], sidx[p], semI[p])
        pltpu.async_copy(dst3.at[s, pl.ds(g * GCH, GCH)], didx[p], semI[p])

    load_idx(jnp.int32(0), 0)

    zero16 = jnp.zeros((16,), jnp.float32)

    def zf(r, _):
        rows[0][r, pl.ds(0, 16)] = zero16
        rows[0][r, pl.ds(16, 16)] = zero16
        return 0

    lax.fori_loop(0, CH, zf, 0)
    base = s * RPT

    def zacc(i, _):
        pltpu.sync_copy(rows[0], acc.at[pl.ds(base + i * CH, CH)])
        return 0

    lax.fori_loop(0, RPT // CH, zacc, 0)
    _REM = RPT - (RPT // CH) * CH
    if _REM:
        pltpu.sync_copy(rows[0].at[pl.ds(0, _REM)],
                        acc.at[pl.ds(base + RPT - _REM, _REM)])
    plsc.subcore_barrier()

    def wait_idx(p):
        pltpu.make_async_copy(src3.at[c, s, pl.ds(0, GCH)], sidx[p],
                              semI[p]).wait()
        pltpu.make_async_copy(dst3.at[s, pl.ds(0, GCH)], didx[p],
                              semI[p]).wait()

    def start_g(slot, p, row, _g=None, _b=None):
        pltpu.async_copy(table.at[sidx[p].at[row]], rows[slot], semG[slot])

    def wait_g(slot):
        pltpu.make_async_copy(table.at[sidx[0].at[0]], rows[slot],
                              semG[slot]).wait()

    def start_s(slot, p, row):
        pltpu.async_copy(rows[slot], acc.at[didx[p].at[row]], semS[slot],
                         add=True)

    def wait_s(slot):
        pltpu.make_async_copy(rows[slot], acc.at[didx[0].at[0]],
                              semS[slot]).wait()

    AHEAD = 5
    SLACK = DEPTH - AHEAD
    _OFF = {0: 0, 1: (8 * 1) % DEPTH, 2: (8 * 2) % DEPTH}

    def group_body(g, pg, pn, first, last):
        off = _OFF[pg]
        if not last:
            load_idx(g + 1, pn)
        for b in range(GCH):
            slot = (off + b) % DEPTH
            s2 = (off + b + AHEAD) % DEPTH
            if b == GCH - AHEAD and not last:
                wait_idx(pn)
            wait_g(slot)
            start_s(slot, pg, b)
            if not (first and b < SLACK):
                wait_s(s2)
            if b < GCH - AHEAD:
                start_g(s2, pg, b + AHEAD)
            elif not last:
                start_g(s2, pn, b - (GCH - AHEAD))

    wait_idx(0)
    for b in range(AHEAD):
        start_g(b, 0, b)
    group_body(jnp.int32(0), 0, 1, first=True, last=False)
    group_body(jnp.int32(1), 1, 2, first=False, last=False)

    def triple(u, _):
        gbase = 3 * u + 2
        group_body(gbase, 2, 0, first=False, last=False)
        group_body(gbase + 1, 0, 1, first=False, last=False)
        group_body(gbase + 2, 1, 2, first=False, last=False)
        return 0

    lax.fori_loop(0, 15, triple, 0)

    group_body(jnp.int32(47), 2, 0, first=False, last=False)
    group_body(jnp.int32(48), 0, 1, first=False, last=False)
    group_body(jnp.int32(49), 1, 2, first=False, last=True)
    for b in range(GCH - SLACK, GCH):
        wait_s((_OFF[1] + b) % DEPTH)

    plsc.subcore_barrier()
    pltpu.sync_copy(acc.at[pl.ds(base, RPT)], out.at[c, pl.ds(base, RPT)])


def _tcA_body(deg_ref, x_ref, w1_ref, p_ref, dinv_ref):
    deg = deg_ref[:, 0] + deg_ref[:, 1] + 1.0
    dinv = 1.0 / jnp.sqrt(deg)
    g = jnp.dot(x_ref[...], w1_ref[...], preferred_element_type=jnp.float32)
    p = g * dinv[:, None]
    p_ref[0] = p[:, :HH]
    p_ref[1] = p[:, HH:]
    dinv_ref[...] = dinv[:, None]


def _tcA(deg2, x, W1):
    return pl.pallas_call(
        _tcA_body,
        grid=(GRID,),
        in_specs=[
            pl.BlockSpec((BR, NC), lambda i: (i, 0)),
            pl.BlockSpec((BR, IN_DIM), lambda i: (i, 0)),
            pl.BlockSpec((IN_DIM, HID), lambda i: (0, 0)),
        ],
        out_specs=[
            pl.BlockSpec((NC, BR, HH), lambda i: (0, i, 0)),
            pl.BlockSpec((BR, 1), lambda i: (i, 0)),
        ],
        out_shape=[
            jax.ShapeDtypeStruct((NC, NPAD, HH), jnp.float32),
            jax.ShapeDtypeStruct((NPAD, 1), jnp.float32),
        ],
    )(deg2, x, W1)


def _tcB_body(acc_ref, p_ref, dinv_ref, w2_ref, b1_ref, p2_ref):
    dinv = dinv_ref[...]
    hl = (acc_ref[0] + p_ref[0]) * dinv + b1_ref[:, :HH]
    hr = (acc_ref[1] + p_ref[1]) * dinv + b1_ref[:, HH:]
    h = jnp.concatenate([hl, hr], axis=1)
    h = jnp.maximum(h, 0.0)
    g2 = jnp.dot(h, w2_ref[...], preferred_element_type=jnp.float32)
    p2 = g2 * dinv
    p2_ref[0] = p2[:, :HH]
    p2_ref[1] = p2[:, HH:]


def _tcB(acc1, p, dinv, W2, b1r):
    return pl.pallas_call(
        _tcB_body,
        grid=(GRID,),
        in_specs=[
            pl.BlockSpec((NC, BR, HH), lambda i: (0, i, 0)),
            pl.BlockSpec((NC, BR, HH), lambda i: (0, i, 0)),
            pl.BlockSpec((BR, 1), lambda i: (i, 0)),
            pl.BlockSpec((HID, HID), lambda i: (0, 0)),
            pl.BlockSpec((1, HID), lambda i: (0, 0)),
        ],
        out_specs=pl.BlockSpec((NC, BR, HH), lambda i: (0, i, 0)),
        out_shape=jax.ShapeDtypeStruct((NC, NPAD, HH), jnp.float32),
    )(acc1, p, dinv, W2, b1r)


def _tcC_body(acc_ref, p2_ref, dinv_ref, whw_ref, b2_ref, bhw_ref, out_ref):
    dinv = dinv_ref[...]
    hl = (acc_ref[0] + p2_ref[0]) * dinv + b2_ref[:, :HH]
    hr = (acc_ref[1] + p2_ref[1]) * dinv + b2_ref[:, HH:]
    h = jnp.concatenate([hl, hr], axis=1)
    h = jnp.maximum(h, 0.0)
    out_ref[...] = (
        jnp.dot(h, whw_ref[...], preferred_element_type=jnp.float32)
        + bhw_ref[...]
    )


def _tcC(acc2, p2, dinv, Whw, b2r, bhw):
    return pl.pallas_call(
        _tcC_body,
        grid=(GRID,),
        in_specs=[
            pl.BlockSpec((NC, BR, HH), lambda i: (0, i, 0)),
            pl.BlockSpec((NC, BR, HH), lambda i: (0, i, 0)),
            pl.BlockSpec((BR, 1), lambda i: (i, 0)),
            pl.BlockSpec((HID, 2), lambda i: (0, 0)),
            pl.BlockSpec((1, HID), lambda i: (0, 0)),
            pl.BlockSpec((1, 2), lambda i: (0, 0)),
        ],
        out_specs=pl.BlockSpec((BR, 2), lambda i: (i, 0)),
        out_shape=jax.ShapeDtypeStruct((N, 2), jnp.float32),
    )(acc2, p2, dinv, Whw, b2r, bhw)


def kernel(x, edge_index, W1, b1, W2, b2, Wo, bo, Ww, bw):
    ei = edge_index.astype(jnp.int32)
    pad = jnp.full((EPAD - E,), TRASH, jnp.int32)
    src_p = jnp.concatenate([ei[0], pad])
    src3 = jnp.stack([src_p, src_p + NPAD]).reshape(NC, NS, CPS, CH)
    dst3 = jnp.concatenate([ei[1], pad]).reshape(NS, CPS, CH)

    deg2 = _deg_kernel(dst3)
    p1, dinv = _tcA(deg2.T, x, W1)
    table1 = p1.reshape(NC * NPAD, HH)
    acc1 = _prop_kernel(table1, src3, dst3)
    p2 = _tcB(acc1, p1, dinv, W2, b1.reshape(1, HID))
    table2 = p2.reshape(NC * NPAD, HH)
    acc2 = _prop_kernel(table2, src3, dst3)
    Whw = jnp.concatenate([Wo, Ww], axis=1)
    bhw = jnp.stack([bo[0], bw[0]]).reshape(1, 2)
    out = _tcC(acc2, p2, dinv, Whw, b2.reshape(1, HID), bhw)
    return out[:, 0], out[:, 1]


# speedup vs baseline: 1.0619x; 1.0006x over previous
import functools

import jax
import jax.numpy as jnp
from jax import lax
from jax.experimental import pallas as pl
from jax.experimental.pallas import tpu as pltpu
from jax.experimental.pallas import tpu_sc as plsc

N = 50000
E = 800000
IN_DIM = 6
HID = 64
HH = HID // 2

NC = 2
NS = 16

CH = 128
GCH = 8
CPS = 400
EPAD = NS * CPS * CH
TRASH = N
NPAD = 50048
RPT = NPAD // NS
NPAD_D = 51200
RPT_D = NPAD_D // NS

BR = 5000
GRID = N // BR

DEPTH = 6

_mesh = plsc.VectorSubcoreMesh(
    core_axis_name="c", subcore_axis_name="s", num_cores=NC, num_subcores=NS
)


DGRP = (CPS // NC) // GCH


@functools.partial(
    pl.kernel,
    out_type=jax.ShapeDtypeStruct((NC, NPAD_D), jnp.float32),
    mesh=_mesh,
    scratch_types=[
        pltpu.VMEM((GCH, CH), jnp.int32),
        pltpu.VMEM((GCH, CH), jnp.int32),
        pltpu.VMEM((NPAD_D,), jnp.float32),
        pltpu.VMEM((RPT_D,), jnp.float32),
        pltpu.VMEM((RPT_D,), jnp.float32),
        pltpu.VMEM_SHARED((NS, NPAD_D), jnp.float32),
        pltpu.SemaphoreType.DMA,
        pltpu.SemaphoreType.DMA,
    ],
    compiler_params=pltpu.CompilerParams(
        use_tc_tiling_on_sc=False, needs_layout_passes=False
    ),
)
def _deg_kernel(dst3, deg_out, didx0, didx1, hist, racc, rtmp, slab,
                semI0, semI1):
    c = lax.axis_index("c")
    s = lax.axis_index("s")
    zero16 = jnp.zeros((16,), jnp.float32)
    one16 = jnp.ones((16,), jnp.float32)
    didx = (didx0, didx1)
    semI = (semI0, semI1)

    def zf(i, _):
        hist[pl.ds(i * 16, 16)] = zero16
        return 0

    lax.fori_loop(0, NPAD_D // 16, zf, 0)

    gbase = c * DGRP

    def load_grp(g, p):
        pltpu.async_copy(dst3.at[s, pl.ds((gbase + g) * GCH, GCH)], didx[p],
                         semI[p])

    def wait_grp(p):
        pltpu.make_async_copy(dst3.at[s, pl.ds(0, GCH)], didx[p],
                              semI[p]).wait()

    def process(p):
        for r in range(GCH):
            for i in range(CH // 16):
                v = didx[p][r, pl.ds(i * 16, 16)]
                plsc.addupdate_scatter(hist, [v], one16)

    load_grp(jnp.int32(0), 0)
    load_grp(jnp.int32(1), 1)

    def hbody(u, _):
        wait_grp(0)
        process(0)
        load_grp(2 * u + 2, 0)
        wait_grp(1)
        process(1)

        @pl.when(u < 11)
        def _():
            load_grp(2 * u + 3, 1)

        return 0

    lax.fori_loop(0, (DGRP - 1) // 2, hbody, 0)
    wait_grp(0)
    process(0)

    pltpu.sync_copy(hist, slab.at[s])
    plsc.subcore_barrier()
    rbase = s * RPT_D
    pltpu.sync_copy(slab.at[0, pl.ds(rbase, RPT_D)], racc)

    def radd(k, _):
        pltpu.sync_copy(slab.at[k, pl.ds(rbase, RPT_D)], rtmp)

        def vadd(i, _):
            racc[pl.ds(i * 16, 16)] = (
                racc[pl.ds(i * 16, 16)] + rtmp[pl.ds(i * 16, 16)]
            )
            return 0

        lax.fori_loop(0, RPT_D // 16, vadd, 0)
        return 0

    lax.fori_loop(1, NS, radd, 0)
    pltpu.sync_copy(racc, deg_out.at[c, pl.ds(rbase, RPT_D)])


@functools.partial(
    pl.kernel,
    out_type=jax.ShapeDtypeStruct((NC, NPAD, HH), jnp.float32),
    mesh=_mesh,
    scratch_types=(
        [pltpu.VMEM((CH, HH), jnp.float32)] * DEPTH
        + [pltpu.VMEM((GCH, CH), jnp.int32)] * 3
        + [pltpu.VMEM((GCH, CH), jnp.int32)] * 3
        + [pltpu.VMEM_SHARED((NPAD, HH), jnp.float32)]
        + [pltpu.SemaphoreType.DMA] * (2 * DEPTH + 3)
    ),
    compiler_params=pltpu.CompilerParams(use_tc_tiling_on_sc=False),
)
def _prop_kernel(table, src3, dst3, out, *refs):
    rows = refs[0:DEPTH]
    sidx = refs[DEPTH:DEPTH + 3]
    didx = refs[DEPTH + 3:DEPTH + 6]
    acc = refs[DEPTH + 6]
    semG = refs[DEPTH + 7:DEPTH + 7 + DEPTH]
    semS = refs[DEPTH + 7 + DEPTH:DEPTH + 7 + 2 * DEPTH]
    semI = refs[DEPTH + 7 + 2 * DEPTH:]
    c = lax.axis_index("c")
    s = lax.axis_index("s")

    def load_idx(g, p):
        pltpu.async_copy(src3.at[c, s, pl.ds(g * GCH, GCH)], sidx[p], semI[p])
        pltpu.async_copy(dst3.at[s, pl.ds(g * GCH, GCH)], didx[p], semI[p])

    load_idx(jnp.int32(0), 0)

    zero16 = jnp.zeros((16,), jnp.float32)

    def zf(r, _):
        rows[0][r, pl.ds(0, 16)] = zero16
        rows[0][r, pl.ds(16, 16)] = zero16
        return 0

    lax.fori_loop(0, CH, zf, 0)
    base = s * RPT

    def zacc(i, _):
        pltpu.sync_copy(rows[0], acc.at[pl.ds(base + i * CH, CH)])
        return 0

    lax.fori_loop(0, RPT // CH, zacc, 0)
    _REM = RPT - (RPT // CH) * CH
    if _REM:
        pltpu.sync_copy(rows[0].at[pl.ds(0, _REM)],
                        acc.at[pl.ds(base + RPT - _REM, _REM)])
    plsc.subcore_barrier()

    def wait_idx(p):
        pltpu.make_async_copy(src3.at[c, s, pl.ds(0, GCH)], sidx[p],
                              semI[p]).wait()
        pltpu.make_async_copy(dst3.at[s, pl.ds(0, GCH)], didx[p],
                              semI[p]).wait()

    def start_g(slot, p, row):
        pltpu.async_copy(table.at[sidx[p].at[row]], rows[slot], semG[slot])

    def wait_g(slot):
        pltpu.make_async_copy(table.at[sidx[0].at[0]], rows[slot],
                              semG[slot]).wait()

    def start_s(slot, p, row):
        pltpu.async_copy(rows[slot], acc.at[didx[p].at[row]], semS[slot],
                         add=True)

    def wait_s(slot):
        pltpu.make_async_copy(rows[slot], acc.at[didx[0].at[0]],
                              semS[slot]).wait()

    AHEAD = 5
    SLACK = DEPTH - AHEAD
    _OFF = {0: 0, 1: (8 * 1) % DEPTH, 2: (8 * 2) % DEPTH}

    def group_body(g, pg, pn, first, last):
        off = _OFF[pg]
        if not last:
            load_idx(g + 1, pn)
        for b in range(GCH):
            slot = (off + b) % DEPTH
            s2 = (off + b + AHEAD) % DEPTH
            if b == GCH - AHEAD and not last:
                wait_idx(pn)
            wait_g(slot)
            start_s(slot, pg, b)
            if not (first and b < SLACK):
                wait_s(s2)
            if b < GCH - AHEAD:
                start_g(s2, pg, b + AHEAD)
            elif not last:
                start_g(s2, pn, b - (GCH - AHEAD))

    wait_idx(0)
    for b in range(AHEAD):
        start_g(b, 0, b)
    group_body(jnp.int32(0), 0, 1, first=True, last=False)
    group_body(jnp.int32(1), 1, 2, first=False, last=False)

    def triple(u, _):
        gbase = 3 * u + 2
        group_body(gbase, 2, 0, first=False, last=False)
        group_body(gbase + 1, 0, 1, first=False, last=False)
        group_body(gbase + 2, 1, 2, first=False, last=False)
        return 0

    lax.fori_loop(0, 15, triple, 0)

    group_body(jnp.int32(47), 2, 0, first=False, last=False)
    group_body(jnp.int32(48), 0, 1, first=False, last=False)
    group_body(jnp.int32(49), 1, 2, first=False, last=True)
    for b in range(GCH - SLACK, GCH):
        wait_s((_OFF[1] + b) % DEPTH)

    plsc.subcore_barrier()
    pltpu.sync_copy(acc.at[pl.ds(base, RPT)], out.at[c, pl.ds(base, RPT)])


def _tcA_body(deg_ref, x_ref, w1_ref, p_ref, dinv_ref):
    deg = deg_ref[:, 0] + deg_ref[:, 1] + 1.0
    dinv = 1.0 / jnp.sqrt(deg)
    g = jnp.dot(x_ref[...], w1_ref[...], preferred_element_type=jnp.float32)
    p = g * dinv[:, None]
    p_ref[0] = p[:, :HH]
    p_ref[1] = p[:, HH:]
    dinv_ref[...] = dinv[:, None]


def _tcA(deg2, x, W1):
    return pl.pallas_call(
        _tcA_body,
        grid=(GRID,),
        in_specs=[
            pl.BlockSpec((BR, NC), lambda i: (i, 0)),
            pl.BlockSpec((BR, IN_DIM), lambda i: (i, 0)),
            pl.BlockSpec((IN_DIM, HID), lambda i: (0, 0)),
        ],
        out_specs=[
            pl.BlockSpec((NC, BR, HH), lambda i: (0, i, 0)),
            pl.BlockSpec((BR, 1), lambda i: (i, 0)),
        ],
        out_shape=[
            jax.ShapeDtypeStruct((NC, NPAD, HH), jnp.float32),
            jax.ShapeDtypeStruct((NPAD, 1), jnp.float32),
        ],
    )(deg2, x, W1)


def _tcB_body(acc_ref, p_ref, dinv_ref, w2_ref, b1_ref, p2_ref):
    dinv = dinv_ref[...]
    hl = (acc_ref[0] + p_ref[0]) * dinv + b1_ref[:, :HH]
    hr = (acc_ref[1] + p_ref[1]) * dinv + b1_ref[:, HH:]
    h = jnp.concatenate([hl, hr], axis=1)
    h = jnp.maximum(h, 0.0)
    g2 = jnp.dot(h, w2_ref[...], preferred_element_type=jnp.float32)
    p2 = g2 * dinv
    p2_ref[0] = p2[:, :HH]
    p2_ref[1] = p2[:, HH:]


def _tcB(acc1, p, dinv, W2, b1r):
    return pl.pallas_call(
        _tcB_body,
        grid=(GRID,),
        in_specs=[
            pl.BlockSpec((NC, BR, HH), lambda i: (0, i, 0)),
            pl.BlockSpec((NC, BR, HH), lambda i: (0, i, 0)),
            pl.BlockSpec((BR, 1), lambda i: (i, 0)),
            pl.BlockSpec((HID, HID), lambda i: (0, 0)),
            pl.BlockSpec((1, HID), lambda i: (0, 0)),
        ],
        out_specs=pl.BlockSpec((NC, BR, HH), lambda i: (0, i, 0)),
        out_shape=jax.ShapeDtypeStruct((NC, NPAD, HH), jnp.float32),
    )(acc1, p, dinv, W2, b1r)


def _tcC_body(acc_ref, p2_ref, dinv_ref, whw_ref, b2_ref, bhw_ref, out_ref):
    dinv = dinv_ref[...]
    hl = (acc_ref[0] + p2_ref[0]) * dinv + b2_ref[:, :HH]
    hr = (acc_ref[1] + p2_ref[1]) * dinv + b2_ref[:, HH:]
    h = jnp.concatenate([hl, hr], axis=1)
    h = jnp.maximum(h, 0.0)
    out_ref[...] = (
        jnp.dot(h, whw_ref[...], preferred_element_type=jnp.float32)
        + bhw_ref[...]
    )


def _tcC(acc2, p2, dinv, Whw, b2r, bhw):
    return pl.pallas_call(
        _tcC_body,
        grid=(GRID,),
        in_specs=[
            pl.BlockSpec((NC, BR, HH), lambda i: (0, i, 0)),
            pl.BlockSpec((NC, BR, HH), lambda i: (0, i, 0)),
            pl.BlockSpec((BR, 1), lambda i: (i, 0)),
            pl.BlockSpec((HID, 2), lambda i: (0, 0)),
            pl.BlockSpec((1, HID), lambda i: (0, 0)),
            pl.BlockSpec((1, 2), lambda i: (0, 0)),
        ],
        out_specs=pl.BlockSpec((BR, 2), lambda i: (i, 0)),
        out_shape=jax.ShapeDtypeStruct((N, 2), jnp.float32),
    )(acc2, p2, dinv, Whw, b2r, bhw)


def kernel(x, edge_index, W1, b1, W2, b2, Wo, bo, Ww, bw):
    ei = edge_index.astype(jnp.int32)
    pad = jnp.full((EPAD - E,), TRASH, jnp.int32)
    src_p = jnp.concatenate([ei[0], pad])
    src3 = jnp.stack([src_p, src_p + NPAD]).reshape(NC, NS, CPS, CH)
    dst3 = jnp.concatenate([ei[1], pad]).reshape(NS, CPS, CH)

    deg2 = _deg_kernel(dst3)
    p1, dinv = _tcA(deg2.T, x, W1)
    table1 = p1.reshape(NC * NPAD, HH)
    acc1 = _prop_kernel(table1, src3, dst3)
    p2 = _tcB(acc1, p1, dinv, W2, b1.reshape(1, HID))
    table2 = p2.reshape(NC * NPAD, HH)
    acc2 = _prop_kernel(table2, src3, dst3)
    Whw = jnp.concatenate([Wo, Ww], axis=1)
    bhw = jnp.stack([bo[0], bw[0]]).reshape(1, 2)
    out = _tcC(acc2, p2, dinv, Whw, b2.reshape(1, HID), bhw)
    return out[:, 0], out[:, 1]
